# baseline (speedup 1.0000x reference)
import functools

import jax
import jax.numpy as jnp
from jax import lax
from jax.experimental import pallas as pl
from jax.experimental.pallas import tpu as pltpu
from jax.experimental.pallas import tpu_sc as plsc

NUM_WORKERS = 32
LANES = 16
CHUNK = 128


def _make_kernel(batch, dim, n_chunks):
  b_per_w = n_chunks * CHUNK
  mesh = plsc.VectorSubcoreMesh(core_axis_name="c", subcore_axis_name="s")

  @functools.partial(
      pl.kernel,
      out_type=jax.ShapeDtypeStruct((batch,), jnp.float32),
      mesh=mesh,
      compiler_params=pltpu.CompilerParams(
          needs_layout_passes=False, use_tc_tiling_on_sc=False),
      scratch_types=[
          pltpu.VMEM((n_chunks, CHUNK), jnp.int32),
          pltpu.VMEM((n_chunks, CHUNK), jnp.int32),
          pltpu.VMEM((b_per_w, dim), jnp.float32),
          pltpu.VMEM((b_per_w, dim), jnp.float32),
          pltpu.VMEM((b_per_w,), jnp.float32),
          pltpu.SemaphoreType.DMA,
          pltpu.SemaphoreType.DMA,
      ],
  )
  def k(u_hbm, i_hbm, ue_hbm, ie_hbm, out_hbm,
        uidx, iidx, urows, irows, outv, usem, isem):
    wid = lax.axis_index("s") * 2 + lax.axis_index("c")
    base = wid * b_per_w

    pltpu.sync_copy(u_hbm.at[wid], uidx)
    pltpu.sync_copy(i_hbm.at[wid], iidx)

    descs = []
    for c in range(n_chunks):
      dst = pl.ds(c * CHUNK, CHUNK)
      descs.append(pltpu.async_copy(ue_hbm.at[uidx.at[c]], urows.at[dst], usem))
      descs.append(pltpu.async_copy(ie_hbm.at[iidx.at[c]], irows.at[dst], isem))
    for d in descs:
      d.wait()

    r_iota = lax.iota(jnp.int32, LANES)

    def block_body(bi, _):
      rows = r_iota + bi * LANES

      def col_body(j, acc):
        col = jnp.full((LANES,), 0, jnp.int32) + j
        uvec = plsc.load_gather(urows, [rows, col])
        ivec = plsc.load_gather(irows, [rows, col])
        return acc + uvec * ivec

      acc = lax.fori_loop(0, dim, col_body, jnp.zeros((LANES,), jnp.float32))
      outv[pl.ds(bi * LANES, LANES)] = acc
      return 0

    lax.fori_loop(0, b_per_w // LANES, block_body, 0)

    pltpu.sync_copy(outv, out_hbm.at[pl.ds(base, b_per_w)])

  return k


def kernel(user, item, user_emb, item_emb):
  batch = user.shape[0]
  dim = user_emb.shape[1]
  b_per_w = batch // NUM_WORKERS
  n_chunks = b_per_w // CHUNK
  u2 = user.astype(jnp.int32).reshape(NUM_WORKERS, n_chunks, CHUNK)
  i2 = item.astype(jnp.int32).reshape(NUM_WORKERS, n_chunks, CHUNK)
  k = _make_kernel(batch, dim, n_chunks)
  return k(u2, i2, user_emb, item_emb)
